# Initial kernel scaffold; baseline (speedup 1.0000x reference)
import jax
import jax.numpy as jnp
from jax.experimental import pallas as pl


def kernel(x, edge_index, edge_weight, graph_ids, W1_first, b1_first, bn1g_first, bn1b_first, W2_first, b2_first, W1_rest, b1_rest, bn1g_rest, bn1b_rest, W2_rest, b2_rest, bng, bnb, eps, streams_weight, bn0_g, bn0_b, prelu_a, W_out, b_out, bn2_g, bn2_b):
    raise NotImplementedError("write your pallas kernel here")


# speedup vs baseline: 7.1008x; 7.1008x over previous
import functools

import jax
import jax.numpy as jnp
from jax import lax
from jax.experimental import pallas as pl
from jax.experimental.pallas import tpu as pltpu
from jax.experimental.pallas import tpu_sc as plsc

N = 10000
E = 320000
D = 128
S = 4
H = 64
L = 3
G = 128
OUT = 128

NSUB = 16
NCORE = 2
CHUNK = 112
NBUF = 3
SUP = 12
ROWS_PER_TILE = 624
_TAIL = N - NSUB * ROWS_PER_TILE
_ZCHUNKS = (112, 112, 112, 112, 112, 64)

_NSUP = -(-(E // NSUB) // (CHUNK * SUP))
EPAD = NSUB * _NSUP * SUP * CHUNK
_NSUP1 = -(-(E // (NCORE * NSUB)) // (CHUNK * SUP))
EPAD1 = NCORE * NSUB * _NSUP1 * SUP * CHUNK


@functools.lru_cache(maxsize=None)
def _make_spmm(edge_split):
    fh = D
    nsup = _NSUP1 if edge_split else _NSUP
    mesh = plsc.VectorSubcoreMesh(core_axis_name="c", subcore_axis_name="s",
                                  num_cores=NCORE, num_subcores=NSUB)

    @functools.partial(
        pl.kernel,
        out_type=jax.ShapeDtypeStruct((NCORE * N, fh), jnp.float32),
        mesh=mesh,
        scratch_types=[
            pltpu.VMEM((SUP, CHUNK), jnp.int32),
            pltpu.VMEM((SUP, CHUNK), jnp.int32),
            pltpu.VMEM((SUP, CHUNK), jnp.float32),
            pltpu.VMEM((CHUNK, fh), jnp.float32),
            pltpu.VMEM((CHUNK, fh), jnp.float32),
            pltpu.VMEM((CHUNK, fh), jnp.float32),
            pltpu.VMEM_SHARED((N, fh), jnp.float32),
            pltpu.SemaphoreType.DMA,
            pltpu.SemaphoreType.DMA,
            pltpu.SemaphoreType.DMA,
            pltpu.SemaphoreType.DMA,
            pltpu.SemaphoreType.DMA,
            pltpu.SemaphoreType.DMA,
        ],
    )
    def spmm(h_hbm, src_hbm, dst_hbm, w_hbm, out_hbm,
             src_v, dst_v, w_v, r0, r1, r2, acc,
             g0, g1, g2, s0, s1, s2):
        rows = (r0, r1, r2)
        gsem = (g0, g1, g2)
        ssem = (s0, s1, s2)
        cid = lax.axis_index("c")
        sid = lax.axis_index("s")
        slab = cid * NSUB + sid if edge_split else sid

        zero = jnp.zeros((16,), jnp.float32)

        def zbody(i, carry):
            for k in range(fh // 16):
                r0[i, pl.ds(k * 16, 16)] = zero
            return carry

        lax.fori_loop(0, CHUNK, zbody, 0)
        zbase = sid * ROWS_PER_TILE
        zoff = 0
        for zc in _ZCHUNKS:
            pltpu.sync_copy(r0.at[pl.ds(0, zc)],
                            acc.at[pl.ds(zbase + zoff, zc)])
            zoff += zc

        @pl.when(sid == NSUB - 1)
        def _():
            pltpu.sync_copy(r0.at[pl.ds(0, _TAIL)],
                            acc.at[pl.ds(NSUB * ROWS_PER_TILE, _TAIL)])

        plsc.subcore_barrier()

        def superbody(so, carry):
            pltpu.sync_copy(src_hbm.at[slab, so], src_v)
            pltpu.sync_copy(dst_hbm.at[slab, so], dst_v)
            pltpu.sync_copy(w_hbm.at[slab, so], w_v)

            if not edge_split:
                off = jnp.full((16,), cid * N, jnp.int32)

                def add_off(i, c2):
                    for k in range(CHUNK // 16):
                        sl = pl.ds(k * 16, 16)
                        src_v[i, sl] = src_v[i, sl] + off
                    return c2

                lax.fori_loop(0, SUP, add_off, 0)

            pltpu.async_copy(h_hbm.at[src_v.at[0]], rows[0], gsem[0])
            pltpu.async_copy(h_hbm.at[src_v.at[1]], rows[1], gsem[1])

            for jc in range(SUP):
                b = jc % NBUF
                pltpu.make_async_copy(
                    h_hbm.at[src_v.at[jc]], rows[b], gsem[b]).wait()

                def scale_body(gi, c2, _b=b, _jc=jc):
                    wg = w_v[_jc, pl.ds(gi * 16, 16)]
                    for e in range(16):
                        w16 = jnp.full((16,), wg[e], jnp.float32)
                        i = gi * 16 + e
                        for k in range(fh // 16):
                            sl = pl.ds(k * 16, 16)
                            rows[_b][i, sl] = rows[_b][i, sl] * w16
                    return c2

                lax.fori_loop(0, CHUNK // 16, scale_body, 0)

                pltpu.async_copy(rows[b], acc.at[dst_v.at[jc]], ssem[b],
                                 add=True)

                jn = jc + 2
                if jn < SUP:
                    bn = jn % NBUF
                    if jn >= NBUF:
                        pltpu.make_async_copy(
                            rows[bn], acc.at[dst_v.at[0]], ssem[bn]).wait()
                    pltpu.async_copy(
                        h_hbm.at[src_v.at[jn]], rows[bn], gsem[bn])

            for b in range(NBUF):
                pltpu.make_async_copy(
                    rows[b], acc.at[dst_v.at[0]], ssem[b]).wait()
            return carry

        lax.fori_loop(0, nsup, superbody, 0)

        plsc.subcore_barrier()

        wbase = sid * ROWS_PER_TILE
        pltpu.sync_copy(acc.at[pl.ds(wbase, ROWS_PER_TILE)],
                        out_hbm.at[pl.ds(cid * N + wbase, ROWS_PER_TILE)])

        @pl.when(sid == NSUB - 1)
        def _():
            tb = NSUB * ROWS_PER_TILE
            pltpu.sync_copy(acc.at[pl.ds(tb, _TAIL)],
                            out_hbm.at[pl.ds(cid * N + tb, _TAIL)])

    return spmm


def _bn(t, g, b):
    m = jnp.mean(t, axis=0, keepdims=True)
    v = jnp.mean((t - m) * (t - m), axis=0, keepdims=True)
    return (t - m) * lax.rsqrt(v + 1e-5) * g + b


def _mlp(hin0, hin1, w1_ref, b1_ref, g1_ref, bb1_ref,
         w2_ref, b2_ref, g2_ref, bb2_ref):
    w1 = w1_ref[...]
    fhalf = w1.shape[0] // 2
    t = (jnp.dot(hin0, w1[:fhalf], preferred_element_type=jnp.float32)
         + jnp.dot(hin1, w1[fhalf:], preferred_element_type=jnp.float32)
         + b1_ref[...])
    t = jnp.maximum(_bn(t, g1_ref[...], bb1_ref[...]), 0.0)
    u = jnp.dot(t, w2_ref[...], preferred_element_type=jnp.float32) + b2_ref[...]
    return jnp.maximum(_bn(u, g2_ref[...], bb2_ref[...]), 0.0)


def _pool(u, gid_ref, swrep_ref):
    gid = gid_ref[...]
    iot = lax.broadcasted_iota(jnp.int32, (G, N), 0)
    p = (iot == gid).astype(jnp.float32)
    pooled = jnp.dot(p, u, preferred_element_type=jnp.float32)
    pw = pooled * swrep_ref[...]
    return (pw[:, 0:H] + pw[:, H:2 * H]
            + pw[:, 2 * H:3 * H] + pw[:, 3 * H:4 * H])


def _layer_body(first):
    def body(h_ref, agg_ref, ceps_ref, w1_ref, b1_ref, g1_ref, bb1_ref,
             w2_ref, b2_ref, g2_ref, bb2_ref, gid_ref, swrep_ref,
             hout_ref, wp_ref):
        ceps = ceps_ref[...]
        fhalf = ceps.shape[1] // 2
        if first:
            agg = agg_ref[0] + agg_ref[1]
            hin0 = h_ref[:, :fhalf] * ceps[:, :fhalf] + agg[:, :fhalf]
            hin1 = h_ref[:, fhalf:] * ceps[:, fhalf:] + agg[:, fhalf:]
        else:
            hin0 = h_ref[0] * ceps[:, :fhalf] + agg_ref[0]
            hin1 = h_ref[1] * ceps[:, fhalf:] + agg_ref[1]
        u = _mlp(hin0, hin1, w1_ref, b1_ref, g1_ref, bb1_ref,
                 w2_ref, b2_ref, g2_ref, bb2_ref)
        hout_ref[0] = u[:, :S * H // 2]
        hout_ref[1] = u[:, S * H // 2:]
        wp_ref[...] = _pool(u, gid_ref, swrep_ref)
    return body


def _layer2_head_body(h_ref, agg_ref, ceps_ref, w1_ref, b1_ref, g1_ref,
                      bb1_ref, w2_ref, b2_ref, g2_ref, bb2_ref, gid_ref,
                      swrep_ref, wp0_ref, wp1_ref, bn0g_ref, bn0b_ref,
                      pa_ref, wout_ref, bout_ref, bn2g_ref, bn2b_ref,
                      z_ref):
    ceps = ceps_ref[...]
    fhalf = ceps.shape[1] // 2
    hin0 = h_ref[0] * ceps[:, :fhalf] + agg_ref[0]
    hin1 = h_ref[1] * ceps[:, fhalf:] + agg_ref[1]
    u = _mlp(hin0, hin1, w1_ref, b1_ref, g1_ref, bb1_ref,
             w2_ref, b2_ref, g2_ref, bb2_ref)
    wp2 = _pool(u, gid_ref, swrep_ref)
    rep = jnp.concatenate([wp0_ref[...], wp1_ref[...], wp2], axis=1)
    z = _bn(rep, bn0g_ref[...], bn0b_ref[...])
    z = jnp.where(z >= 0.0, z, z * pa_ref[...])
    z = jnp.dot(z, wout_ref[...], preferred_element_type=jnp.float32) \
        + bout_ref[...]
    z_ref[...] = _bn(z, bn2g_ref[...], bn2b_ref[...])


_layer_call = {}
for _first in (True, False):
    _layer_call[_first] = pl.pallas_call(
        _layer_body(_first),
        out_shape=(jax.ShapeDtypeStruct((2, N, S * H // 2), jnp.float32),
                   jax.ShapeDtypeStruct((G, H), jnp.float32)),
    )

_head_call = pl.pallas_call(
    _layer2_head_body,
    out_shape=jax.ShapeDtypeStruct((G, OUT), jnp.float32),
)


def kernel(x, edge_index, edge_weight, graph_ids, W1_first, b1_first,
           bn1g_first, bn1b_first, W2_first, b2_first, W1_rest, b1_rest,
           bn1g_rest, bn1b_rest, W2_rest, b2_rest, bng, bnb, eps,
           streams_weight, bn0_g, bn0_b, prelu_a, W_out, b_out, bn2_g,
           bn2_b):
    Ds = D // S
    f32 = jnp.float32

    def slabs(a, zero, epad, ntiles, nsup):
        a = jnp.concatenate([a, jnp.full((epad - E,), zero, a.dtype)])
        return a.reshape(ntiles, nsup, SUP, CHUNK)

    src = slabs(edge_index[0], 0, EPAD, NSUB, _NSUP)
    dst = slabs(edge_index[1], 0, EPAD, NSUB, _NSUP)
    wgt = slabs(edge_weight, 0.0, EPAD, NSUB, _NSUP)
    src1 = slabs(edge_index[0], 0, EPAD1, NCORE * NSUB, _NSUP1)
    dst1 = slabs(edge_index[1], 0, EPAD1, NCORE * NSUB, _NSUP1)
    wgt1 = slabs(edge_weight, 0.0, EPAD1, NCORE * NSUB, _NSUP1)

    bd = jax.scipy.linalg.block_diag
    w1 = [bd(*[W1_first[s] for s in range(S)])]
    w2 = [bd(*[W2_first[s] for s in range(S)])]
    b1 = [b1_first.reshape(1, S * H)]
    b2 = [b2_first.reshape(1, S * H)]
    g1 = [bn1g_first.reshape(1, S * H)]
    bb1 = [bn1b_first.reshape(1, S * H)]
    for l in range(1, L):
        w1.append(bd(*[W1_rest[l - 1, s] for s in range(S)]))
        w2.append(bd(*[W2_rest[l - 1, s] for s in range(S)]))
        b1.append(b1_rest[l - 1].reshape(1, S * H))
        b2.append(b2_rest[l - 1].reshape(1, S * H))
        g1.append(bn1g_rest[l - 1].reshape(1, S * H))
        bb1.append(bn1b_rest[l - 1].reshape(1, S * H))
    g2 = [bng[l].reshape(1, S * H) for l in range(L)]
    bb2 = [bnb[l].reshape(1, S * H) for l in range(L)]
    ceps = [jnp.repeat(1.0 + eps[0], Ds)[None, :]]
    for l in range(1, L):
        ceps.append(jnp.repeat(1.0 + eps[l], H)[None, :])
    swrep = jnp.repeat(streams_weight.reshape(S), H)[None, :]
    gid2d = graph_ids.reshape(1, N)

    agg0 = _make_spmm(True)(x, src1, dst1, wgt1).reshape(2, N, D)
    h1, wp0 = _layer_call[True](
        x, agg0, ceps[0], w1[0], b1[0], g1[0], bb1[0],
        w2[0], b2[0], g2[0], bb2[0], gid2d, swrep)

    agg1 = _make_spmm(False)(h1.reshape(2 * N, S * H // 2), src, dst, wgt)
    agg1 = agg1.reshape(2, N, S * H // 2)
    h2, wp1 = _layer_call[False](
        h1, agg1, ceps[1], w1[1], b1[1], g1[1], bb1[1],
        w2[1], b2[1], g2[1], bb2[1], gid2d, swrep)

    agg2 = _make_spmm(False)(h2.reshape(2 * N, S * H // 2), src, dst, wgt)
    agg2 = agg2.reshape(2, N, S * H // 2)
    z = _head_call(
        h2, agg2, ceps[2], w1[2], b1[2], g1[2], bb1[2],
        w2[2], b2[2], g2[2], bb2[2], gid2d, swrep, wp0, wp1,
        bn0_g.reshape(1, -1), bn0_b.reshape(1, -1), prelu_a.reshape(1, 1),
        W_out, b_out.reshape(1, -1), bn2_g.reshape(1, -1),
        bn2_b.reshape(1, -1))
    return z


# speedup vs baseline: 13.2102x; 1.8604x over previous
import functools

import jax
import jax.numpy as jnp
from jax import lax
from jax.experimental import pallas as pl
from jax.experimental.pallas import tpu as pltpu
from jax.experimental.pallas import tpu_sc as plsc

N = 10000
E = 320000
D = 128
S = 4
H = 64
L = 3
G = 128
OUT = 128

NSUB = 16
NCORE = 2
CHUNK = 112
NBUF = 3
SUP = 15
ROWS_PER_TILE = 624
_TAIL = N - NSUB * ROWS_PER_TILE
_ZCHUNKS = (112, 112, 112, 112, 112, 64)

_NSUP = -(-(E // NSUB) // (CHUNK * SUP))
EPAD = NSUB * _NSUP * SUP * CHUNK
_NSUP1 = -(-(E // (NCORE * NSUB)) // (CHUNK * SUP))
EPAD1 = NCORE * NSUB * _NSUP1 * SUP * CHUNK


@functools.lru_cache(maxsize=None)
def _make_spmm(edge_split):
    fh = D
    nsup = _NSUP1 if edge_split else _NSUP
    mesh = plsc.VectorSubcoreMesh(core_axis_name="c", subcore_axis_name="s",
                                  num_cores=NCORE, num_subcores=NSUB)

    @functools.partial(
        pl.kernel,
        out_type=jax.ShapeDtypeStruct((NCORE * N, fh), jnp.float32),
        mesh=mesh,
        scratch_types=[
            pltpu.VMEM((SUP, CHUNK), jnp.int32),
            pltpu.VMEM((SUP, CHUNK), jnp.int32),
            pltpu.VMEM((SUP, CHUNK), jnp.float32),
            pltpu.VMEM((CHUNK, fh), jnp.float32),
            pltpu.VMEM((CHUNK, fh), jnp.float32),
            pltpu.VMEM((CHUNK, fh), jnp.float32),
            pltpu.VMEM_SHARED((N, fh), jnp.float32),
            pltpu.SemaphoreType.DMA,
            pltpu.SemaphoreType.DMA,
            pltpu.SemaphoreType.DMA,
            pltpu.SemaphoreType.DMA,
            pltpu.SemaphoreType.DMA,
            pltpu.SemaphoreType.DMA,
        ],
    )
    def spmm(h_hbm, src_hbm, dst_hbm, w_hbm, out_hbm,
             src_v, dst_v, w_v, r0, r1, r2, acc,
             g0, g1, g2, s0, s1, s2):
        rows = (r0, r1, r2)
        gsem = (g0, g1, g2)
        ssem = (s0, s1, s2)
        cid = lax.axis_index("c")
        sid = lax.axis_index("s")
        slab = cid * NSUB + sid if edge_split else sid

        zero = jnp.zeros((16,), jnp.float32)

        def zbody(i, carry):
            for k in range(fh // 16):
                r0[i, pl.ds(k * 16, 16)] = zero
            return carry

        lax.fori_loop(0, CHUNK, zbody, 0)
        zbase = sid * ROWS_PER_TILE
        zoff = 0
        for zc in _ZCHUNKS:
            pltpu.sync_copy(r0.at[pl.ds(0, zc)],
                            acc.at[pl.ds(zbase + zoff, zc)])
            zoff += zc

        @pl.when(sid == NSUB - 1)
        def _():
            pltpu.sync_copy(r0.at[pl.ds(0, _TAIL)],
                            acc.at[pl.ds(NSUB * ROWS_PER_TILE, _TAIL)])

        plsc.subcore_barrier()

        def superbody(so, carry):
            pltpu.sync_copy(src_hbm.at[slab, so], src_v)
            pltpu.sync_copy(dst_hbm.at[slab, so], dst_v)
            pltpu.sync_copy(w_hbm.at[slab, so], w_v)

            off = jnp.full((16,), cid * N, jnp.int32)

            def add_off(i, c2):
                for k in range(CHUNK // 16):
                    sl = pl.ds(k * 16, 16)
                    src_v[i, sl] = src_v[i, sl] + off
                return c2

            lax.fori_loop(0, SUP, add_off, 0)

            pltpu.async_copy(h_hbm.at[src_v.at[0]], rows[0], gsem[0])
            pltpu.async_copy(h_hbm.at[src_v.at[1]], rows[1], gsem[1])

            for jc in range(SUP):
                b = jc % NBUF
                pltpu.make_async_copy(
                    h_hbm.at[src_v.at[jc]], rows[b], gsem[b]).wait()

                def scale_body(gi, c2, _b=b, _jc=jc):
                    wg = w_v[_jc, pl.ds(gi * 16, 16)]
                    for e in range(16):
                        w16 = jnp.full((16,), wg[e], jnp.float32)
                        i = gi * 16 + e
                        for k in range(fh // 16):
                            sl = pl.ds(k * 16, 16)
                            rows[_b][i, sl] = rows[_b][i, sl] * w16
                    return c2

                lax.fori_loop(0, CHUNK // 16, scale_body, 0)

                pltpu.async_copy(rows[b], acc.at[dst_v.at[jc]], ssem[b],
                                 add=True)

                jn = jc + 2
                if jn < SUP:
                    bn = jn % NBUF
                    if jn >= NBUF:
                        pltpu.make_async_copy(
                            rows[bn], acc.at[dst_v.at[0]], ssem[bn]).wait()
                    pltpu.async_copy(
                        h_hbm.at[src_v.at[jn]], rows[bn], gsem[bn])

            for b in range(NBUF):
                pltpu.make_async_copy(
                    rows[b], acc.at[dst_v.at[0]], ssem[b]).wait()
            return carry

        lax.fori_loop(0, nsup, superbody, 0)

        plsc.subcore_barrier()

        wbase = sid * ROWS_PER_TILE
        pltpu.sync_copy(acc.at[pl.ds(wbase, ROWS_PER_TILE)],
                        out_hbm.at[pl.ds(cid * N + wbase, ROWS_PER_TILE)])

        @pl.when(sid == NSUB - 1)
        def _():
            tb = NSUB * ROWS_PER_TILE
            pltpu.sync_copy(acc.at[pl.ds(tb, _TAIL)],
                            out_hbm.at[pl.ds(cid * N + tb, _TAIL)])

    return spmm


def _bn(t, g, b):
    m = jnp.mean(t, axis=0, keepdims=True)
    v = jnp.mean((t - m) * (t - m), axis=0, keepdims=True)
    return (t - m) * lax.rsqrt(v + 1e-5) * g + b


def _mlp(hin0, hin1, w1_ref, b1_ref, g1_ref, bb1_ref,
         w2_ref, b2_ref, g2_ref, bb2_ref):
    w1 = w1_ref[...]
    fhalf = w1.shape[0] // 2
    t = (jnp.dot(hin0, w1[:fhalf], preferred_element_type=jnp.float32)
         + jnp.dot(hin1, w1[fhalf:], preferred_element_type=jnp.float32)
         + b1_ref[...])
    t = jnp.maximum(_bn(t, g1_ref[...], bb1_ref[...]), 0.0)
    u = jnp.dot(t, w2_ref[...], preferred_element_type=jnp.float32) + b2_ref[...]
    return jnp.maximum(_bn(u, g2_ref[...], bb2_ref[...]), 0.0)


def _pool(u, gid_ref, swrep_ref):
    gid = gid_ref[...]
    iot = lax.broadcasted_iota(jnp.int32, (G, N), 0)
    p = (iot == gid).astype(jnp.float32)
    pooled = jnp.dot(p, u, preferred_element_type=jnp.float32)
    pw = pooled * swrep_ref[...]
    return (pw[:, 0:H] + pw[:, H:2 * H]
            + pw[:, 2 * H:3 * H] + pw[:, 3 * H:4 * H])


def _layer_body(first):
    def body(h_ref, agg_ref, ceps_ref, w1_ref, b1_ref, g1_ref, bb1_ref,
             w2_ref, b2_ref, g2_ref, bb2_ref, gid_ref, swrep_ref,
             hout_ref, wp_ref):
        ceps = ceps_ref[...]
        fhalf = ceps.shape[1] // 2
        if first:
            agg = agg_ref[0] + agg_ref[1]
            hin0 = h_ref[:, :fhalf] * ceps[:, :fhalf] + agg[:, :fhalf]
            hin1 = h_ref[:, fhalf:] * ceps[:, fhalf:] + agg[:, fhalf:]
        else:
            hin0 = h_ref[0] * ceps[:, :fhalf] + agg_ref[0]
            hin1 = h_ref[1] * ceps[:, fhalf:] + agg_ref[1]
        u = _mlp(hin0, hin1, w1_ref, b1_ref, g1_ref, bb1_ref,
                 w2_ref, b2_ref, g2_ref, bb2_ref)
        hout_ref[0] = u[:, :S * H // 2]
        hout_ref[1] = u[:, S * H // 2:]
        wp_ref[...] = _pool(u, gid_ref, swrep_ref)
    return body


def _layer2_head_body(h_ref, agg_ref, ceps_ref, w1_ref, b1_ref, g1_ref,
                      bb1_ref, w2_ref, b2_ref, g2_ref, bb2_ref, gid_ref,
                      swrep_ref, wp0_ref, wp1_ref, bn0g_ref, bn0b_ref,
                      pa_ref, wout_ref, bout_ref, bn2g_ref, bn2b_ref,
                      z_ref):
    ceps = ceps_ref[...]
    fhalf = ceps.shape[1] // 2
    hin0 = h_ref[0] * ceps[:, :fhalf] + agg_ref[0]
    hin1 = h_ref[1] * ceps[:, fhalf:] + agg_ref[1]
    u = _mlp(hin0, hin1, w1_ref, b1_ref, g1_ref, bb1_ref,
             w2_ref, b2_ref, g2_ref, bb2_ref)
    wp2 = _pool(u, gid_ref, swrep_ref)
    rep = jnp.concatenate([wp0_ref[...], wp1_ref[...], wp2], axis=1)
    z = _bn(rep, bn0g_ref[...], bn0b_ref[...])
    z = jnp.where(z >= 0.0, z, z * pa_ref[...])
    z = jnp.dot(z, wout_ref[...], preferred_element_type=jnp.float32) \
        + bout_ref[...]
    z_ref[...] = _bn(z, bn2g_ref[...], bn2b_ref[...])


_layer_call = {}
for _first in (True, False):
    _layer_call[_first] = pl.pallas_call(
        _layer_body(_first),
        out_shape=(jax.ShapeDtypeStruct((2, N, S * H // 2), jnp.float32),
                   jax.ShapeDtypeStruct((G, H), jnp.float32)),
    )

_head_call = pl.pallas_call(
    _layer2_head_body,
    out_shape=jax.ShapeDtypeStruct((G, OUT), jnp.float32),
)


def kernel(x, edge_index, edge_weight, graph_ids, W1_first, b1_first,
           bn1g_first, bn1b_first, W2_first, b2_first, W1_rest, b1_rest,
           bn1g_rest, bn1b_rest, W2_rest, b2_rest, bng, bnb, eps,
           streams_weight, bn0_g, bn0_b, prelu_a, W_out, b_out, bn2_g,
           bn2_b):
    Ds = D // S
    f32 = jnp.float32

    def slabs(a, zero, epad, ntiles, nsup):
        a = jnp.concatenate([a, jnp.full((epad - E,), zero, a.dtype)])
        return a.reshape(ntiles, nsup, SUP, CHUNK)

    src = slabs(edge_index[0], 0, EPAD, NSUB, _NSUP)
    dst = slabs(edge_index[1], 0, EPAD, NSUB, _NSUP)
    wgt = slabs(edge_weight, 0.0, EPAD, NSUB, _NSUP)
    src1 = slabs(edge_index[0], 0, EPAD1, NCORE * NSUB, _NSUP1)
    dst1 = slabs(edge_index[1], 0, EPAD1, NCORE * NSUB, _NSUP1)
    wgt1 = slabs(edge_weight, 0.0, EPAD1, NCORE * NSUB, _NSUP1)

    bd = jax.scipy.linalg.block_diag
    w1 = [bd(*[W1_first[s] for s in range(S)])]
    w2 = [bd(*[W2_first[s] for s in range(S)])]
    b1 = [b1_first.reshape(1, S * H)]
    b2 = [b2_first.reshape(1, S * H)]
    g1 = [bn1g_first.reshape(1, S * H)]
    bb1 = [bn1b_first.reshape(1, S * H)]
    for l in range(1, L):
        w1.append(bd(*[W1_rest[l - 1, s] for s in range(S)]))
        w2.append(bd(*[W2_rest[l - 1, s] for s in range(S)]))
        b1.append(b1_rest[l - 1].reshape(1, S * H))
        b2.append(b2_rest[l - 1].reshape(1, S * H))
        g1.append(bn1g_rest[l - 1].reshape(1, S * H))
        bb1.append(bn1b_rest[l - 1].reshape(1, S * H))
    g2 = [bng[l].reshape(1, S * H) for l in range(L)]
    bb2 = [bnb[l].reshape(1, S * H) for l in range(L)]
    ceps = [jnp.repeat(1.0 + eps[0], Ds)[None, :]]
    for l in range(1, L):
        ceps.append(jnp.repeat(1.0 + eps[l], H)[None, :])
    swrep = jnp.repeat(streams_weight.reshape(S), H)[None, :]
    gid2d = graph_ids.reshape(1, N)

    xx = jnp.concatenate([x, x], axis=0)
    agg0 = _make_spmm(True)(xx, src1, dst1, wgt1).reshape(2, N, D)
    h1, wp0 = _layer_call[True](
        x, agg0, ceps[0], w1[0], b1[0], g1[0], bb1[0],
        w2[0], b2[0], g2[0], bb2[0], gid2d, swrep)

    agg1 = _make_spmm(False)(h1.reshape(2 * N, S * H // 2), src, dst, wgt)
    agg1 = agg1.reshape(2, N, S * H // 2)
    h2, wp1 = _layer_call[False](
        h1, agg1, ceps[1], w1[1], b1[1], g1[1], bb1[1],
        w2[1], b2[1], g2[1], bb2[1], gid2d, swrep)

    agg2 = _make_spmm(False)(h2.reshape(2 * N, S * H // 2), src, dst, wgt)
    agg2 = agg2.reshape(2, N, S * H // 2)
    z = _head_call(
        h2, agg2, ceps[2], w1[2], b1[2], g1[2], bb1[2],
        w2[2], b2[2], g2[2], bb2[2], gid2d, swrep, wp0, wp1,
        bn0_g.reshape(1, -1), bn0_b.reshape(1, -1), prelu_a.reshape(1, 1),
        W_out, b_out.reshape(1, -1), bn2_g.reshape(1, -1),
        bn2_b.reshape(1, -1))
    return z


# speedup vs baseline: 13.4127x; 1.0153x over previous
import functools

import jax
import jax.numpy as jnp
from jax import lax
from jax.experimental import pallas as pl
from jax.experimental.pallas import tpu as pltpu
from jax.experimental.pallas import tpu_sc as plsc

N = 10000
E = 320000
D = 128
S = 4
H = 64
L = 3
G = 128
OUT = 128

NSUB = 16
NCORE = 2
CHUNK = 80
NBUF = 4
SUP = 21
ROWS_PER_TILE = 624
_TAIL = N - NSUB * ROWS_PER_TILE
_ZCHUNKS = (80, 80, 80, 80, 80, 80, 80, 64)

_NSUP = -(-(E // NSUB) // (CHUNK * SUP))
EPAD = NSUB * _NSUP * SUP * CHUNK
_NSUP1 = -(-(E // (NCORE * NSUB)) // (CHUNK * SUP))
EPAD1 = NCORE * NSUB * _NSUP1 * SUP * CHUNK


@functools.lru_cache(maxsize=None)
def _make_spmm(edge_split):
    fh = D
    nsup = _NSUP1 if edge_split else _NSUP
    mesh = plsc.VectorSubcoreMesh(core_axis_name="c", subcore_axis_name="s",
                                  num_cores=NCORE, num_subcores=NSUB)

    @functools.partial(
        pl.kernel,
        out_type=jax.ShapeDtypeStruct((NCORE * N, fh), jnp.float32),
        mesh=mesh,
        scratch_types=[
            pltpu.VMEM((SUP, CHUNK), jnp.int32),
            pltpu.VMEM((SUP, CHUNK), jnp.int32),
            pltpu.VMEM((SUP, CHUNK), jnp.float32),
            pltpu.VMEM((CHUNK, fh), jnp.float32),
            pltpu.VMEM((CHUNK, fh), jnp.float32),
            pltpu.VMEM((CHUNK, fh), jnp.float32),
            pltpu.VMEM((CHUNK, fh), jnp.float32),
            pltpu.VMEM_SHARED((N, fh), jnp.float32),
            pltpu.SemaphoreType.DMA,
            pltpu.SemaphoreType.DMA,
            pltpu.SemaphoreType.DMA,
            pltpu.SemaphoreType.DMA,
            pltpu.SemaphoreType.DMA,
            pltpu.SemaphoreType.DMA,
            pltpu.SemaphoreType.DMA,
            pltpu.SemaphoreType.DMA,
        ],
    )
    def spmm(h_hbm, src_hbm, dst_hbm, w_hbm, out_hbm,
             src_v, dst_v, w_v, r0, r1, r2, r3, acc,
             g0, g1, g2, g3, s0, s1, s2, s3):
        rows = (r0, r1, r2, r3)
        gsem = (g0, g1, g2, g3)
        ssem = (s0, s1, s2, s3)
        cid = lax.axis_index("c")
        sid = lax.axis_index("s")
        slab = cid * NSUB + sid if edge_split else sid

        zero = jnp.zeros((16,), jnp.float32)

        def zbody(i, carry):
            for k in range(fh // 16):
                r0[i, pl.ds(k * 16, 16)] = zero
            return carry

        lax.fori_loop(0, CHUNK, zbody, 0)
        zbase = sid * ROWS_PER_TILE
        zoff = 0
        for zc in _ZCHUNKS:
            pltpu.sync_copy(r0.at[pl.ds(0, zc)],
                            acc.at[pl.ds(zbase + zoff, zc)])
            zoff += zc

        @pl.when(sid == NSUB - 1)
        def _():
            pltpu.sync_copy(r0.at[pl.ds(0, _TAIL)],
                            acc.at[pl.ds(NSUB * ROWS_PER_TILE, _TAIL)])

        plsc.subcore_barrier()

        def superbody(so, carry):
            pltpu.sync_copy(src_hbm.at[slab, so], src_v)
            pltpu.sync_copy(dst_hbm.at[slab, so], dst_v)
            pltpu.sync_copy(w_hbm.at[slab, so], w_v)

            off = jnp.full((16,), cid * N, jnp.int32)

            def add_off(i, c2):
                for k in range(CHUNK // 16):
                    sl = pl.ds(k * 16, 16)
                    src_v[i, sl] = src_v[i, sl] + off
                return c2

            lax.fori_loop(0, SUP, add_off, 0)

            pltpu.async_copy(h_hbm.at[src_v.at[0]], rows[0], gsem[0])
            pltpu.async_copy(h_hbm.at[src_v.at[1]], rows[1], gsem[1])

            for jc in range(SUP):
                b = jc % NBUF
                pltpu.make_async_copy(
                    h_hbm.at[src_v.at[jc]], rows[b], gsem[b]).wait()

                def scale_body(gi, c2, _b=b, _jc=jc):
                    wg = w_v[_jc, pl.ds(gi * 16, 16)]
                    for e in range(16):
                        w16 = jnp.full((16,), wg[e], jnp.float32)
                        i = gi * 16 + e
                        for k in range(fh // 16):
                            sl = pl.ds(k * 16, 16)
                            rows[_b][i, sl] = rows[_b][i, sl] * w16
                    return c2

                lax.fori_loop(0, CHUNK // 16, scale_body, 0)

                pltpu.async_copy(rows[b], acc.at[dst_v.at[jc]], ssem[b],
                                 add=True)

                jn = jc + 2
                if jn < SUP:
                    bn = jn % NBUF
                    if jn >= NBUF:
                        pltpu.make_async_copy(
                            rows[bn], acc.at[dst_v.at[0]], ssem[bn]).wait()
                    pltpu.async_copy(
                        h_hbm.at[src_v.at[jn]], rows[bn], gsem[bn])

            for b in range(NBUF):
                pltpu.make_async_copy(
                    rows[b], acc.at[dst_v.at[0]], ssem[b]).wait()
            return carry

        lax.fori_loop(0, nsup, superbody, 0)

        plsc.subcore_barrier()

        wbase = sid * ROWS_PER_TILE
        pltpu.sync_copy(acc.at[pl.ds(wbase, ROWS_PER_TILE)],
                        out_hbm.at[pl.ds(cid * N + wbase, ROWS_PER_TILE)])

        @pl.when(sid == NSUB - 1)
        def _():
            tb = NSUB * ROWS_PER_TILE
            pltpu.sync_copy(acc.at[pl.ds(tb, _TAIL)],
                            out_hbm.at[pl.ds(cid * N + tb, _TAIL)])

    return spmm


def _bn(t, g, b):
    m = jnp.mean(t, axis=0, keepdims=True)
    v = jnp.mean((t - m) * (t - m), axis=0, keepdims=True)
    return (t - m) * lax.rsqrt(v + 1e-5) * g + b


def _mlp(hin0, hin1, w1_ref, b1_ref, g1_ref, bb1_ref,
         w2_ref, b2_ref, g2_ref, bb2_ref):
    w1 = w1_ref[...]
    fhalf = w1.shape[0] // 2
    t = (jnp.dot(hin0, w1[:fhalf], preferred_element_type=jnp.float32)
         + jnp.dot(hin1, w1[fhalf:], preferred_element_type=jnp.float32)
         + b1_ref[...])
    t = jnp.maximum(_bn(t, g1_ref[...], bb1_ref[...]), 0.0)
    u = jnp.dot(t, w2_ref[...], preferred_element_type=jnp.float32) + b2_ref[...]
    return jnp.maximum(_bn(u, g2_ref[...], bb2_ref[...]), 0.0)


def _pool(u, gid_ref, swrep_ref):
    gid = gid_ref[...]
    iot = lax.broadcasted_iota(jnp.int32, (G, N), 0)
    p = (iot == gid).astype(jnp.float32)
    pooled = jnp.dot(p, u, preferred_element_type=jnp.float32)
    pw = pooled * swrep_ref[...]
    return (pw[:, 0:H] + pw[:, H:2 * H]
            + pw[:, 2 * H:3 * H] + pw[:, 3 * H:4 * H])


def _layer_body(first):
    def body(h_ref, agg_ref, ceps_ref, w1_ref, b1_ref, g1_ref, bb1_ref,
             w2_ref, b2_ref, g2_ref, bb2_ref, gid_ref, swrep_ref,
             hout_ref, wp_ref):
        ceps = ceps_ref[...]
        fhalf = ceps.shape[1] // 2
        if first:
            agg = agg_ref[0] + agg_ref[1]
            hin0 = h_ref[:, :fhalf] * ceps[:, :fhalf] + agg[:, :fhalf]
            hin1 = h_ref[:, fhalf:] * ceps[:, fhalf:] + agg[:, fhalf:]
        else:
            hin0 = h_ref[0] * ceps[:, :fhalf] + agg_ref[0]
            hin1 = h_ref[1] * ceps[:, fhalf:] + agg_ref[1]
        u = _mlp(hin0, hin1, w1_ref, b1_ref, g1_ref, bb1_ref,
                 w2_ref, b2_ref, g2_ref, bb2_ref)
        hout_ref[0] = u[:, :S * H // 2]
        hout_ref[1] = u[:, S * H // 2:]
        wp_ref[...] = _pool(u, gid_ref, swrep_ref)
    return body


def _layer2_head_body(h_ref, agg_ref, ceps_ref, w1_ref, b1_ref, g1_ref,
                      bb1_ref, w2_ref, b2_ref, g2_ref, bb2_ref, gid_ref,
                      swrep_ref, wp0_ref, wp1_ref, bn0g_ref, bn0b_ref,
                      pa_ref, wout_ref, bout_ref, bn2g_ref, bn2b_ref,
                      z_ref):
    ceps = ceps_ref[...]
    fhalf = ceps.shape[1] // 2
    hin0 = h_ref[0] * ceps[:, :fhalf] + agg_ref[0]
    hin1 = h_ref[1] * ceps[:, fhalf:] + agg_ref[1]
    u = _mlp(hin0, hin1, w1_ref, b1_ref, g1_ref, bb1_ref,
             w2_ref, b2_ref, g2_ref, bb2_ref)
    wp2 = _pool(u, gid_ref, swrep_ref)
    rep = jnp.concatenate([wp0_ref[...], wp1_ref[...], wp2], axis=1)
    z = _bn(rep, bn0g_ref[...], bn0b_ref[...])
    z = jnp.where(z >= 0.0, z, z * pa_ref[...])
    z = jnp.dot(z, wout_ref[...], preferred_element_type=jnp.float32) \
        + bout_ref[...]
    z_ref[...] = _bn(z, bn2g_ref[...], bn2b_ref[...])


_layer_call = {}
for _first in (True, False):
    _layer_call[_first] = pl.pallas_call(
        _layer_body(_first),
        out_shape=(jax.ShapeDtypeStruct((2, N, S * H // 2), jnp.float32),
                   jax.ShapeDtypeStruct((G, H), jnp.float32)),
    )

_head_call = pl.pallas_call(
    _layer2_head_body,
    out_shape=jax.ShapeDtypeStruct((G, OUT), jnp.float32),
)


def kernel(x, edge_index, edge_weight, graph_ids, W1_first, b1_first,
           bn1g_first, bn1b_first, W2_first, b2_first, W1_rest, b1_rest,
           bn1g_rest, bn1b_rest, W2_rest, b2_rest, bng, bnb, eps,
           streams_weight, bn0_g, bn0_b, prelu_a, W_out, b_out, bn2_g,
           bn2_b):
    Ds = D // S
    f32 = jnp.float32

    def slabs(a, zero, epad, ntiles, nsup):
        a = jnp.concatenate([a, jnp.full((epad - E,), zero, a.dtype)])
        return a.reshape(ntiles, nsup, SUP, CHUNK)

    src = slabs(edge_index[0], 0, EPAD, NSUB, _NSUP)
    dst = slabs(edge_index[1], 0, EPAD, NSUB, _NSUP)
    wgt = slabs(edge_weight, 0.0, EPAD, NSUB, _NSUP)
    src1 = slabs(edge_index[0], 0, EPAD1, NCORE * NSUB, _NSUP1)
    dst1 = slabs(edge_index[1], 0, EPAD1, NCORE * NSUB, _NSUP1)
    wgt1 = slabs(edge_weight, 0.0, EPAD1, NCORE * NSUB, _NSUP1)

    bd = jax.scipy.linalg.block_diag
    w1 = [bd(*[W1_first[s] for s in range(S)])]
    w2 = [bd(*[W2_first[s] for s in range(S)])]
    b1 = [b1_first.reshape(1, S * H)]
    b2 = [b2_first.reshape(1, S * H)]
    g1 = [bn1g_first.reshape(1, S * H)]
    bb1 = [bn1b_first.reshape(1, S * H)]
    for l in range(1, L):
        w1.append(bd(*[W1_rest[l - 1, s] for s in range(S)]))
        w2.append(bd(*[W2_rest[l - 1, s] for s in range(S)]))
        b1.append(b1_rest[l - 1].reshape(1, S * H))
        b2.append(b2_rest[l - 1].reshape(1, S * H))
        g1.append(bn1g_rest[l - 1].reshape(1, S * H))
        bb1.append(bn1b_rest[l - 1].reshape(1, S * H))
    g2 = [bng[l].reshape(1, S * H) for l in range(L)]
    bb2 = [bnb[l].reshape(1, S * H) for l in range(L)]
    ceps = [jnp.repeat(1.0 + eps[0], Ds)[None, :]]
    for l in range(1, L):
        ceps.append(jnp.repeat(1.0 + eps[l], H)[None, :])
    swrep = jnp.repeat(streams_weight.reshape(S), H)[None, :]
    gid2d = graph_ids.reshape(1, N)

    xx = jnp.concatenate([x, x], axis=0)
    agg0 = _make_spmm(True)(xx, src1, dst1, wgt1).reshape(2, N, D)
    h1, wp0 = _layer_call[True](
        x, agg0, ceps[0], w1[0], b1[0], g1[0], bb1[0],
        w2[0], b2[0], g2[0], bb2[0], gid2d, swrep)

    agg1 = _make_spmm(False)(h1.reshape(2 * N, S * H // 2), src, dst, wgt)
    agg1 = agg1.reshape(2, N, S * H // 2)
    h2, wp1 = _layer_call[False](
        h1, agg1, ceps[1], w1[1], b1[1], g1[1], bb1[1],
        w2[1], b2[1], g2[1], bb2[1], gid2d, swrep)

    agg2 = _make_spmm(False)(h2.reshape(2 * N, S * H // 2), src, dst, wgt)
    agg2 = agg2.reshape(2, N, S * H // 2)
    z = _head_call(
        h2, agg2, ceps[2], w1[2], b1[2], g1[2], bb1[2],
        w2[2], b2[2], g2[2], bb2[2], gid2d, swrep, wp0, wp1,
        bn0_g.reshape(1, -1), bn0_b.reshape(1, -1), prelu_a.reshape(1, 1),
        W_out, b_out.reshape(1, -1), bn2_g.reshape(1, -1),
        bn2_b.reshape(1, -1))
    return z


# speedup vs baseline: 13.4516x; 1.0029x over previous
import functools

import jax
import jax.numpy as jnp
from jax import lax
from jax.experimental import pallas as pl
from jax.experimental.pallas import tpu as pltpu
from jax.experimental.pallas import tpu_sc as plsc

N = 10000
E = 320000
D = 128
S = 4
H = 64
L = 3
G = 128
OUT = 128

NSUB = 16
NCORE = 2
CHUNK = 80
NBUF = 4
SUP = 21
ROWS_PER_TILE = 624
_TAIL = N - NSUB * ROWS_PER_TILE
_ZCHUNKS = (80, 80, 80, 80, 80, 80, 80, 64)

_NSUP = -(-(E // NSUB) // (CHUNK * SUP))
EPAD = NSUB * _NSUP * SUP * CHUNK
_NSUP1 = -(-(E // (NCORE * NSUB)) // (CHUNK * SUP))
EPAD1 = NCORE * NSUB * _NSUP1 * SUP * CHUNK


@functools.lru_cache(maxsize=None)
def _make_spmm(edge_split):
    fh = D
    nsup = _NSUP1 if edge_split else _NSUP
    mesh = plsc.VectorSubcoreMesh(core_axis_name="c", subcore_axis_name="s",
                                  num_cores=NCORE, num_subcores=NSUB)

    @functools.partial(
        pl.kernel,
        out_type=jax.ShapeDtypeStruct((NCORE * N, fh), jnp.float32),
        mesh=mesh,
        scratch_types=[
            pltpu.VMEM((SUP, CHUNK), jnp.int32),
            pltpu.VMEM((SUP, CHUNK), jnp.int32),
            pltpu.VMEM((SUP, CHUNK), jnp.float32),
            pltpu.VMEM((CHUNK, fh), jnp.float32),
            pltpu.VMEM((CHUNK, fh), jnp.float32),
            pltpu.VMEM((CHUNK, fh), jnp.float32),
            pltpu.VMEM((CHUNK, fh), jnp.float32),
            pltpu.VMEM_SHARED((N, fh), jnp.float32),
            pltpu.SemaphoreType.DMA,
            pltpu.SemaphoreType.DMA,
            pltpu.SemaphoreType.DMA,
            pltpu.SemaphoreType.DMA,
            pltpu.SemaphoreType.DMA,
            pltpu.SemaphoreType.DMA,
            pltpu.SemaphoreType.DMA,
            pltpu.SemaphoreType.DMA,
        ],
    )
    def spmm(h_hbm, src_hbm, dst_hbm, w_hbm, out_hbm,
             src_v, dst_v, w_v, r0, r1, r2, r3, acc,
             g0, g1, g2, g3, s0, s1, s2, s3):
        rows = (r0, r1, r2, r3)
        gsem = (g0, g1, g2, g3)
        ssem = (s0, s1, s2, s3)
        cid = lax.axis_index("c")
        sid = lax.axis_index("s")
        slab = cid * NSUB + sid if edge_split else sid

        zero = jnp.zeros((16,), jnp.float32)

        def zbody(i, carry):
            for k in range(fh // 16):
                r0[i, pl.ds(k * 16, 16)] = zero
            return carry

        lax.fori_loop(0, CHUNK, zbody, 0)
        zbase = sid * ROWS_PER_TILE
        zoff = 0
        for zi, zc in enumerate(_ZCHUNKS):
            pltpu.async_copy(r0.at[pl.ds(0, zc)],
                             acc.at[pl.ds(zbase + zoff, zc)],
                             gsem[zi % NBUF])
            zoff += zc

        @pl.when(sid == NSUB - 1)
        def _():
            pltpu.async_copy(r0.at[pl.ds(0, _TAIL)],
                             acc.at[pl.ds(NSUB * ROWS_PER_TILE, _TAIL)],
                             ssem[0])

        zoff = 0
        for zi, zc in enumerate(_ZCHUNKS):
            pltpu.make_async_copy(r0.at[pl.ds(0, zc)],
                                  acc.at[pl.ds(zbase + zoff, zc)],
                                  gsem[zi % NBUF]).wait()
            zoff += zc

        @pl.when(sid == NSUB - 1)
        def _():
            pltpu.make_async_copy(r0.at[pl.ds(0, _TAIL)],
                                  acc.at[pl.ds(NSUB * ROWS_PER_TILE, _TAIL)],
                                  ssem[0]).wait()

        plsc.subcore_barrier()

        def superbody(so, carry):
            pltpu.sync_copy(src_hbm.at[slab, so], src_v)
            pltpu.sync_copy(dst_hbm.at[slab, so], dst_v)
            pltpu.sync_copy(w_hbm.at[slab, so], w_v)

            off = jnp.full((16,), cid * N, jnp.int32)

            def add_off(i, c2):
                for k in range(CHUNK // 16):
                    sl = pl.ds(k * 16, 16)
                    src_v[i, sl] = src_v[i, sl] + off
                return c2

            lax.fori_loop(0, SUP, add_off, 0)

            pltpu.async_copy(h_hbm.at[src_v.at[0]], rows[0], gsem[0])
            pltpu.async_copy(h_hbm.at[src_v.at[1]], rows[1], gsem[1])

            for jc in range(SUP):
                b = jc % NBUF
                pltpu.make_async_copy(
                    h_hbm.at[src_v.at[jc]], rows[b], gsem[b]).wait()

                def scale_body(gi, c2, _b=b, _jc=jc):
                    wg = w_v[_jc, pl.ds(gi * 16, 16)]
                    for e in range(16):
                        w16 = jnp.full((16,), wg[e], jnp.float32)
                        i = gi * 16 + e
                        for k in range(fh // 16):
                            sl = pl.ds(k * 16, 16)
                            rows[_b][i, sl] = rows[_b][i, sl] * w16
                    return c2

                lax.fori_loop(0, CHUNK // 16, scale_body, 0)

                pltpu.async_copy(rows[b], acc.at[dst_v.at[jc]], ssem[b],
                                 add=True)

                jn = jc + 2
                if jn < SUP:
                    bn = jn % NBUF
                    if jn >= NBUF:
                        pltpu.make_async_copy(
                            rows[bn], acc.at[dst_v.at[0]], ssem[bn]).wait()
                    pltpu.async_copy(
                        h_hbm.at[src_v.at[jn]], rows[bn], gsem[bn])

            for b in range(NBUF):
                pltpu.make_async_copy(
                    rows[b], acc.at[dst_v.at[0]], ssem[b]).wait()
            return carry

        lax.fori_loop(0, nsup, superbody, 0)

        plsc.subcore_barrier()

        wbase = sid * ROWS_PER_TILE
        pltpu.sync_copy(acc.at[pl.ds(wbase, ROWS_PER_TILE)],
                        out_hbm.at[pl.ds(cid * N + wbase, ROWS_PER_TILE)])

        @pl.when(sid == NSUB - 1)
        def _():
            tb = NSUB * ROWS_PER_TILE
            pltpu.sync_copy(acc.at[pl.ds(tb, _TAIL)],
                            out_hbm.at[pl.ds(cid * N + tb, _TAIL)])

    return spmm


def _bn(t, g, b):
    m = jnp.mean(t, axis=0, keepdims=True)
    v = jnp.mean((t - m) * (t - m), axis=0, keepdims=True)
    return (t - m) * lax.rsqrt(v + 1e-5) * g + b


def _mlp(hin0, hin1, w1_ref, b1_ref, g1_ref, bb1_ref,
         w2_ref, b2_ref, g2_ref, bb2_ref):
    w1 = w1_ref[...]
    fhalf = w1.shape[0] // 2
    t = (jnp.dot(hin0, w1[:fhalf], preferred_element_type=jnp.float32)
         + jnp.dot(hin1, w1[fhalf:], preferred_element_type=jnp.float32)
         + b1_ref[...])
    t = jnp.maximum(_bn(t, g1_ref[...], bb1_ref[...]), 0.0)
    u = jnp.dot(t, w2_ref[...], preferred_element_type=jnp.float32) + b2_ref[...]
    return jnp.maximum(_bn(u, g2_ref[...], bb2_ref[...]), 0.0)


def _pool(u, gid_ref, swrep_ref):
    gid = gid_ref[...]
    iot = lax.broadcasted_iota(jnp.int32, (G, N), 0)
    p = (iot == gid).astype(jnp.float32)
    pooled = jnp.dot(p, u, preferred_element_type=jnp.float32)
    pw = pooled * swrep_ref[...]
    return (pw[:, 0:H] + pw[:, H:2 * H]
            + pw[:, 2 * H:3 * H] + pw[:, 3 * H:4 * H])


def _layer_body(first):
    def body(h_ref, agg_ref, ceps_ref, w1_ref, b1_ref, g1_ref, bb1_ref,
             w2_ref, b2_ref, g2_ref, bb2_ref, gid_ref, swrep_ref,
             hout_ref, wp_ref):
        ceps = ceps_ref[...]
        fhalf = ceps.shape[1] // 2
        if first:
            agg = agg_ref[0] + agg_ref[1]
            hin0 = h_ref[:, :fhalf] * ceps[:, :fhalf] + agg[:, :fhalf]
            hin1 = h_ref[:, fhalf:] * ceps[:, fhalf:] + agg[:, fhalf:]
        else:
            hin0 = h_ref[0] * ceps[:, :fhalf] + agg_ref[0]
            hin1 = h_ref[1] * ceps[:, fhalf:] + agg_ref[1]
        u = _mlp(hin0, hin1, w1_ref, b1_ref, g1_ref, bb1_ref,
                 w2_ref, b2_ref, g2_ref, bb2_ref)
        hout_ref[0] = u[:, :S * H // 2]
        hout_ref[1] = u[:, S * H // 2:]
        wp_ref[...] = _pool(u, gid_ref, swrep_ref)
    return body


def _layer2_head_body(h_ref, agg_ref, ceps_ref, w1_ref, b1_ref, g1_ref,
                      bb1_ref, w2_ref, b2_ref, g2_ref, bb2_ref, gid_ref,
                      swrep_ref, wp0_ref, wp1_ref, bn0g_ref, bn0b_ref,
                      pa_ref, wout_ref, bout_ref, bn2g_ref, bn2b_ref,
                      z_ref):
    ceps = ceps_ref[...]
    fhalf = ceps.shape[1] // 2
    hin0 = h_ref[0] * ceps[:, :fhalf] + agg_ref[0]
    hin1 = h_ref[1] * ceps[:, fhalf:] + agg_ref[1]
    u = _mlp(hin0, hin1, w1_ref, b1_ref, g1_ref, bb1_ref,
             w2_ref, b2_ref, g2_ref, bb2_ref)
    wp2 = _pool(u, gid_ref, swrep_ref)
    rep = jnp.concatenate([wp0_ref[...], wp1_ref[...], wp2], axis=1)
    z = _bn(rep, bn0g_ref[...], bn0b_ref[...])
    z = jnp.where(z >= 0.0, z, z * pa_ref[...])
    z = jnp.dot(z, wout_ref[...], preferred_element_type=jnp.float32) \
        + bout_ref[...]
    z_ref[...] = _bn(z, bn2g_ref[...], bn2b_ref[...])


_layer_call = {}
for _first in (True, False):
    _layer_call[_first] = pl.pallas_call(
        _layer_body(_first),
        out_shape=(jax.ShapeDtypeStruct((2, N, S * H // 2), jnp.float32),
                   jax.ShapeDtypeStruct((G, H), jnp.float32)),
    )

_head_call = pl.pallas_call(
    _layer2_head_body,
    out_shape=jax.ShapeDtypeStruct((G, OUT), jnp.float32),
)


def kernel(x, edge_index, edge_weight, graph_ids, W1_first, b1_first,
           bn1g_first, bn1b_first, W2_first, b2_first, W1_rest, b1_rest,
           bn1g_rest, bn1b_rest, W2_rest, b2_rest, bng, bnb, eps,
           streams_weight, bn0_g, bn0_b, prelu_a, W_out, b_out, bn2_g,
           bn2_b):
    Ds = D // S
    f32 = jnp.float32

    def slabs(a, zero, epad, ntiles, nsup):
        a = jnp.concatenate([a, jnp.full((epad - E,), zero, a.dtype)])
        return a.reshape(ntiles, nsup, SUP, CHUNK)

    src = slabs(edge_index[0], 0, EPAD, NSUB, _NSUP)
    dst = slabs(edge_index[1], 0, EPAD, NSUB, _NSUP)
    wgt = slabs(edge_weight, 0.0, EPAD, NSUB, _NSUP)
    src1 = slabs(edge_index[0], 0, EPAD1, NCORE * NSUB, _NSUP1)
    dst1 = slabs(edge_index[1], 0, EPAD1, NCORE * NSUB, _NSUP1)
    wgt1 = slabs(edge_weight, 0.0, EPAD1, NCORE * NSUB, _NSUP1)

    bd = jax.scipy.linalg.block_diag
    w1 = [bd(*[W1_first[s] for s in range(S)])]
    w2 = [bd(*[W2_first[s] for s in range(S)])]
    b1 = [b1_first.reshape(1, S * H)]
    b2 = [b2_first.reshape(1, S * H)]
    g1 = [bn1g_first.reshape(1, S * H)]
    bb1 = [bn1b_first.reshape(1, S * H)]
    for l in range(1, L):
        w1.append(bd(*[W1_rest[l - 1, s] for s in range(S)]))
        w2.append(bd(*[W2_rest[l - 1, s] for s in range(S)]))
        b1.append(b1_rest[l - 1].reshape(1, S * H))
        b2.append(b2_rest[l - 1].reshape(1, S * H))
        g1.append(bn1g_rest[l - 1].reshape(1, S * H))
        bb1.append(bn1b_rest[l - 1].reshape(1, S * H))
    g2 = [bng[l].reshape(1, S * H) for l in range(L)]
    bb2 = [bnb[l].reshape(1, S * H) for l in range(L)]
    ceps = [jnp.repeat(1.0 + eps[0], Ds)[None, :]]
    for l in range(1, L):
        ceps.append(jnp.repeat(1.0 + eps[l], H)[None, :])
    swrep = jnp.repeat(streams_weight.reshape(S), H)[None, :]
    gid2d = graph_ids.reshape(1, N)

    xx = jnp.concatenate([x, x], axis=0)
    agg0 = _make_spmm(True)(xx, src1, dst1, wgt1).reshape(2, N, D)
    h1, wp0 = _layer_call[True](
        x, agg0, ceps[0], w1[0], b1[0], g1[0], bb1[0],
        w2[0], b2[0], g2[0], bb2[0], gid2d, swrep)

    agg1 = _make_spmm(False)(h1.reshape(2 * N, S * H // 2), src, dst, wgt)
    agg1 = agg1.reshape(2, N, S * H // 2)
    h2, wp1 = _layer_call[False](
        h1, agg1, ceps[1], w1[1], b1[1], g1[1], bb1[1],
        w2[1], b2[1], g2[1], bb2[1], gid2d, swrep)

    agg2 = _make_spmm(False)(h2.reshape(2 * N, S * H // 2), src, dst, wgt)
    agg2 = agg2.reshape(2, N, S * H // 2)
    z = _head_call(
        h2, agg2, ceps[2], w1[2], b1[2], g1[2], bb1[2],
        w2[2], b2[2], g2[2], bb2[2], gid2d, swrep, wp0, wp1,
        bn0_g.reshape(1, -1), bn0_b.reshape(1, -1), prelu_a.reshape(1, 1),
        W_out, b_out.reshape(1, -1), bn2_g.reshape(1, -1),
        bn2_b.reshape(1, -1))
    return z
